# Initial kernel scaffold; baseline (speedup 1.0000x reference)
import jax
import jax.numpy as jnp
from jax.experimental import pallas as pl


def kernel(row_indices, col_indices, row_table, col_table):
    raise NotImplementedError("write your pallas kernel here")


# speedup vs baseline: 5.8187x; 5.8187x over previous
import functools

import jax
import jax.numpy as jnp
from jax import lax
from jax.experimental import pallas as pl
from jax.experimental.pallas import tpu as pltpu
from jax.experimental.pallas import tpu_sc as plsc

B = 4096
L = 200
D = 64
N = B * L
NC = 2
NS = 16
NW = NC * NS
PER_W = N // NW
C = 512
KCH = C // 128
CHUNKS = PER_W // C
IDX_ROWS_PER_W = PER_W // 128


def _body(ridx_hbm, cidx_hbm, rowt_hbm, colt_hbm, out_hbm,
          ridx_v, cidx_v, rowbuf, colbuf, sem):
    wid = lax.axis_index("s") * NC + lax.axis_index("c")
    idx_row_base = wid * IDX_ROWS_PER_W

    def chunk(i, carry):
        ib = idx_row_base + i * KCH
        pltpu.sync_copy(ridx_hbm.at[pl.ds(ib, KCH)], ridx_v)
        pltpu.sync_copy(cidx_hbm.at[pl.ds(ib, KCH)], cidx_v)
        cps = []
        for j in range(KCH):
            cps.append(pltpu.async_copy(
                rowt_hbm.at[ridx_v.at[j]], rowbuf.at[pl.ds(j * 128, 128)], sem))
            cps.append(pltpu.async_copy(
                colt_hbm.at[cidx_v.at[j]], colbuf.at[pl.ds(j * 128, 128)], sem))
        for cp in cps:
            cp.wait()

        def addrow(l, c2):
            for d in range(D // 16):
                v = colbuf[l, pl.ds(d * 16, 16)]
                plsc.addupdate(rowbuf.at[l, pl.ds(d * 16, 16)], v)
            return c2

        lax.fori_loop(0, C, addrow, 0)
        pltpu.sync_copy(rowbuf, out_hbm.at[pl.ds(wid * PER_W + i * C, C)])
        return carry

    lax.fori_loop(0, CHUNKS, chunk, 0)


@jax.jit
def kernel(row_indices, col_indices, row_table, col_table):
    ridx = row_indices.reshape(N // 128, 128).astype(jnp.int32)
    cidx = col_indices.reshape(N // 128, 128).astype(jnp.int32)
    k = pl.kernel(
        _body,
        mesh=plsc.VectorSubcoreMesh(core_axis_name="c", subcore_axis_name="s"),
        compiler_params=pltpu.CompilerParams(use_tc_tiling_on_sc=False),
        out_type=jax.ShapeDtypeStruct((N, D), jnp.float32),
        scratch_types=[
            pltpu.VMEM((KCH, 128), jnp.int32),
            pltpu.VMEM((KCH, 128), jnp.int32),
            pltpu.VMEM((C, D), jnp.float32),
            pltpu.VMEM((C, D), jnp.float32),
            pltpu.SemaphoreType.DMA,
        ],
    )
    out = k(ridx, cidx, row_table, col_table)
    return out.reshape(B, L, D)


# speedup vs baseline: 6.0940x; 1.0473x over previous
import functools

import jax
import jax.numpy as jnp
from jax import lax
from jax.experimental import pallas as pl
from jax.experimental.pallas import tpu as pltpu
from jax.experimental.pallas import tpu_sc as plsc

B = 4096
L = 200
D = 64
N = B * L
NC = 2
NS = 16
NW = NC * NS
PER_W = N // NW
C = 512
KCH = C // 128
CHUNKS = PER_W // C
IDX_ROWS_PER_W = PER_W // 128


def _body(ridx_hbm, cidx_hbm, rowt_hbm, colt_hbm, out_hbm,
          ridx_v, cidx_v, rowbuf, sem):
    wid = lax.axis_index("s") * NC + lax.axis_index("c")
    idx_row_base = wid * IDX_ROWS_PER_W

    def chunk(i, carry):
        ib = idx_row_base + i * KCH
        pltpu.sync_copy(ridx_hbm.at[pl.ds(ib, KCH)], ridx_v)
        pltpu.sync_copy(cidx_hbm.at[pl.ds(ib, KCH)], cidx_v)
        cps = []
        for j in range(KCH):
            cps.append(pltpu.async_copy(
                rowt_hbm.at[ridx_v.at[j]], rowbuf.at[pl.ds(j * 128, 128)], sem))
        for cp in cps:
            cp.wait()
        cps = []
        for j in range(KCH):
            cps.append(pltpu.async_copy(
                colt_hbm.at[cidx_v.at[j]], rowbuf.at[pl.ds(j * 128, 128)], sem,
                add=True))
        for cp in cps:
            cp.wait()
        pltpu.sync_copy(rowbuf, out_hbm.at[pl.ds(wid * PER_W + i * C, C)])
        return carry

    lax.fori_loop(0, CHUNKS, chunk, 0)


@jax.jit
def kernel(row_indices, col_indices, row_table, col_table):
    ridx = row_indices.reshape(N // 128, 128).astype(jnp.int32)
    cidx = col_indices.reshape(N // 128, 128).astype(jnp.int32)
    k = pl.kernel(
        _body,
        mesh=plsc.VectorSubcoreMesh(core_axis_name="c", subcore_axis_name="s"),
        compiler_params=pltpu.CompilerParams(use_tc_tiling_on_sc=False),
        out_type=jax.ShapeDtypeStruct((N, D), jnp.float32),
        scratch_types=[
            pltpu.VMEM((KCH, 128), jnp.int32),
            pltpu.VMEM((KCH, 128), jnp.int32),
            pltpu.VMEM((C, D), jnp.float32),
            pltpu.SemaphoreType.DMA,
        ],
    )
    out = k(ridx, cidx, row_table, col_table)
    return out.reshape(B, L, D)


# speedup vs baseline: 6.1633x; 1.0114x over previous
import functools

import jax
import jax.numpy as jnp
from jax import lax
from jax.experimental import pallas as pl
from jax.experimental.pallas import tpu as pltpu
from jax.experimental.pallas import tpu_sc as plsc

B = 4096
L = 200
D = 64
N = B * L
NC = 2
NS = 16
NW = NC * NS
PER_W = N // NW
C = 256
KCH = C // 128
CHUNKS = PER_W // C
IDX_ROWS_PER_W = PER_W // 128
RING = 4


def _body(ridx_hbm, cidx_hbm, rowt_hbm, colt_hbm, out_hbm,
          ridx_v, cidx_v, rowbuf, semidx, semr, semc, semout):
    wid = lax.axis_index("s") * NC + lax.axis_index("c")
    idx_row_base = wid * IDX_ROWS_PER_W
    out_base = wid * PER_W

    def idx_cps(i, s):
        ib = idx_row_base + i * KCH
        return [
            pltpu.make_async_copy(ridx_hbm.at[pl.ds(ib, KCH)], ridx_v.at[s], semidx),
            pltpu.make_async_copy(cidx_hbm.at[pl.ds(ib, KCH)], cidx_v.at[s], semidx),
        ]

    def row_cps(s):
        return [
            pltpu.make_async_copy(
                rowt_hbm.at[ridx_v.at[s, j]],
                rowbuf.at[s, pl.ds(j * 128, 128)], semr)
            for j in range(KCH)
        ]

    def col_cps(s):
        return [
            pltpu.make_async_copy(
                colt_hbm.at[cidx_v.at[s, j]],
                rowbuf.at[s, pl.ds(j * 128, 128)], semc)
            for j in range(KCH)
        ]

    def out_cp(i, s):
        return pltpu.make_async_copy(
            rowbuf.at[s], out_hbm.at[pl.ds(out_base + i * C, C)], semout)

    def fire(cps, **kw):
        for cp in cps:
            cp.start(**kw)

    def drain(cps):
        for cp in cps:
            cp.wait()

    fire(idx_cps(0, 0))
    drain(idx_cps(0, 0))
    fire(row_cps(0))
    fire(idx_cps(1, 1))

    def step(i, b, has_next, has_next2, has_prev_out):
        drain(row_cps(b))
        fire(col_cps(b), add=True)
        if has_next:
            drain(idx_cps(i, b))
        if has_prev_out:
            drain([out_cp(i, b)])
        if has_next:
            fire(row_cps((b + 1) % RING))
        if has_next2:
            fire(idx_cps(i + 2, (b + 2) % RING))
        drain(col_cps(b))
        fire([out_cp(i, b)])

    step(0, 0, True, True, False)
    step(1, 1, True, True, False)

    def outer(io, carry):
        for b in range(RING):
            i = 2 + io * RING + b
            step(i, (2 + b) % RING, True, True, True)
        return carry

    lax.fori_loop(0, (CHUNKS - 4) // RING, outer, 0)
    step(CHUNKS - 2, (CHUNKS - 2) % RING, True, False, True)
    step(CHUNKS - 1, (CHUNKS - 1) % RING, False, False, True)
    drain([out_cp(CHUNKS - 2, (CHUNKS - 2) % RING)])
    drain([out_cp(CHUNKS - 1, (CHUNKS - 1) % RING)])


@jax.jit
def kernel(row_indices, col_indices, row_table, col_table):
    ridx = row_indices.reshape(N // 128, 128).astype(jnp.int32)
    cidx = col_indices.reshape(N // 128, 128).astype(jnp.int32)
    k = pl.kernel(
        _body,
        mesh=plsc.VectorSubcoreMesh(core_axis_name="c", subcore_axis_name="s"),
        compiler_params=pltpu.CompilerParams(use_tc_tiling_on_sc=False),
        out_type=jax.ShapeDtypeStruct((N, D), jnp.float32),
        scratch_types=[
            pltpu.VMEM((RING, KCH, 128), jnp.int32),
            pltpu.VMEM((RING, KCH, 128), jnp.int32),
            pltpu.VMEM((RING, C, D), jnp.float32),
            pltpu.SemaphoreType.DMA,
            pltpu.SemaphoreType.DMA,
            pltpu.SemaphoreType.DMA,
            pltpu.SemaphoreType.DMA,
        ],
    )
    out = k(ridx, cidx, row_table, col_table)
    return out.reshape(B, L, D)


# speedup vs baseline: 8.2327x; 1.3358x over previous
import functools

import jax
import jax.numpy as jnp
from jax import lax
from jax.experimental import pallas as pl
from jax.experimental.pallas import tpu as pltpu
from jax.experimental.pallas import tpu_sc as plsc

B = 4096
L = 200
D = 64
N = B * L
NC = 2
NS = 16
NW = NC * NS
PER_W = N // NW
C = 256
KCH = C // 128
CHUNKS = PER_W // C
IDX_ROWS_PER_W = PER_W // 128
RING = 4


def _body(ridx_hbm, cidx_hbm, rowt_hbm, colt_hbm, out_hbm,
          ridx_v, cidx_v, rowbuf, rowt_sh, colt_sh, semidx, semr, semc, semout):
    sid = lax.axis_index("s")
    wid = sid * NC + lax.axis_index("c")
    idx_row_base = wid * IDX_ROWS_PER_W
    out_base = wid * PER_W

    @pl.when(sid == 0)
    def _stage_tables():
        pltpu.sync_copy(rowt_hbm, rowt_sh)
        pltpu.sync_copy(colt_hbm, colt_sh)
    plsc.subcore_barrier()

    def idx_cps(i, s):
        ib = idx_row_base + i * KCH
        return [
            pltpu.make_async_copy(ridx_hbm.at[pl.ds(ib, KCH)], ridx_v.at[s], semidx),
            pltpu.make_async_copy(cidx_hbm.at[pl.ds(ib, KCH)], cidx_v.at[s], semidx),
        ]

    def row_cps(s):
        return [
            pltpu.make_async_copy(
                rowt_sh.at[ridx_v.at[s, j]],
                rowbuf.at[s, pl.ds(j * 128, 128)], semr)
            for j in range(KCH)
        ]

    def col_cps(s):
        return [
            pltpu.make_async_copy(
                colt_sh.at[cidx_v.at[s, j]],
                rowbuf.at[s, pl.ds(j * 128, 128)], semc)
            for j in range(KCH)
        ]

    def out_cp(i, s):
        return pltpu.make_async_copy(
            rowbuf.at[s], out_hbm.at[pl.ds(out_base + i * C, C)], semout)

    def fire(cps, **kw):
        for cp in cps:
            cp.start(**kw)

    def drain(cps):
        for cp in cps:
            cp.wait()

    fire(idx_cps(0, 0))
    drain(idx_cps(0, 0))
    fire(row_cps(0))
    fire(idx_cps(1, 1))

    def step(i, b, has_next, has_next2, has_prev_out):
        drain(row_cps(b))
        fire(col_cps(b), add=True)
        if has_next:
            drain(idx_cps(i, b))
        if has_prev_out:
            drain([out_cp(i, b)])
        if has_next:
            fire(row_cps((b + 1) % RING))
        if has_next2:
            fire(idx_cps(i + 2, (b + 2) % RING))
        drain(col_cps(b))
        fire([out_cp(i, b)])

    step(0, 0, True, True, False)
    step(1, 1, True, True, False)

    def outer(io, carry):
        for b in range(RING):
            i = 2 + io * RING + b
            step(i, (2 + b) % RING, True, True, True)
        return carry

    lax.fori_loop(0, (CHUNKS - 4) // RING, outer, 0)
    step(CHUNKS - 2, (CHUNKS - 2) % RING, True, False, True)
    step(CHUNKS - 1, (CHUNKS - 1) % RING, False, False, True)
    drain([out_cp(CHUNKS - 2, (CHUNKS - 2) % RING)])
    drain([out_cp(CHUNKS - 1, (CHUNKS - 1) % RING)])


@jax.jit
def kernel(row_indices, col_indices, row_table, col_table):
    ridx = row_indices.reshape(N // 128, 128).astype(jnp.int32)
    cidx = col_indices.reshape(N // 128, 128).astype(jnp.int32)
    k = pl.kernel(
        _body,
        mesh=plsc.VectorSubcoreMesh(core_axis_name="c", subcore_axis_name="s"),
        compiler_params=pltpu.CompilerParams(use_tc_tiling_on_sc=False),
        out_type=jax.ShapeDtypeStruct((N, D), jnp.float32),
        scratch_types=[
            pltpu.VMEM((RING, KCH, 128), jnp.int32),
            pltpu.VMEM((RING, KCH, 128), jnp.int32),
            pltpu.VMEM((RING, C, D), jnp.float32),
            pltpu.VMEM_SHARED((1000, D), jnp.float32),
            pltpu.VMEM_SHARED((1000, D), jnp.float32),
            pltpu.SemaphoreType.DMA,
            pltpu.SemaphoreType.DMA,
            pltpu.SemaphoreType.DMA,
            pltpu.SemaphoreType.DMA,
        ],
    )
    out = k(ridx, cidx, row_table, col_table)
    return out.reshape(B, L, D)


# speedup vs baseline: 8.2402x; 1.0009x over previous
import functools

import jax
import jax.numpy as jnp
from jax import lax
from jax.experimental import pallas as pl
from jax.experimental.pallas import tpu as pltpu
from jax.experimental.pallas import tpu_sc as plsc

B = 4096
L = 200
D = 64
N = B * L
NC = 2
NS = 16
NW = NC * NS
BAND = B // NW
PH = 2
BPH = BAND // PH
RING = 4
L0 = 128
L1 = L - L0


def _body(ridx_hbm, cidx_hbm, rowt_hbm, colt_hbm, out_hbm,
          rblk, cblk, rT, cT, outbuf, rowt_sh, colt_sh,
          semr, semc, semout):
    sid = lax.axis_index("s")
    wid = sid * NC + lax.axis_index("c")
    band0 = wid * BAND

    @pl.when(sid == 0)
    def _stage_tables():
        pltpu.sync_copy(rowt_hbm, rowt_sh)
        pltpu.sync_copy(colt_hbm, colt_sh)
    plsc.subcore_barrier()

    lanes = lax.iota(jnp.int32, 16)

    def transpose_block(src, dst):
        def trow(l, carry):
            lv = jnp.full((16,), l, jnp.int32)
            for g in range(BPH // 16):
                v = src[l, pl.ds(g * 16, 16)]
                plsc.store_scatter(dst, [lanes + (g * 16), lv], v)
            return carry
        lax.fori_loop(0, L, trow, 0)

    def row_cps(b, s):
        return [
            pltpu.make_async_copy(
                rowt_sh.at[rT.at[b, pl.ds(0, L0)]],
                outbuf.at[pl.ds(s * L, L0)], semr),
            pltpu.make_async_copy(
                rowt_sh.at[rT.at[b, pl.ds(L0, L1)]],
                outbuf.at[pl.ds(s * L + L0, L1)], semr),
        ]

    def col_cps(b, s):
        return [
            pltpu.make_async_copy(
                colt_sh.at[cT.at[b, pl.ds(0, L0)]],
                outbuf.at[pl.ds(s * L, L0)], semc),
            pltpu.make_async_copy(
                colt_sh.at[cT.at[b, pl.ds(L0, L1)]],
                outbuf.at[pl.ds(s * L + L0, L1)], semc),
        ]

    def out_cp(bg, s):
        return pltpu.make_async_copy(
            outbuf.at[pl.ds(s * L, L)],
            out_hbm.at[pl.ds((band0 + bg) * L, L)], semout)

    def fire(cps, **kw):
        for cp in cps:
            cp.start(**kw)

    def drain(cps):
        for cp in cps:
            cp.wait()

    for ph in range(PH):
        col0 = band0 + ph * BPH
        pltpu.sync_copy(ridx_hbm.at[pl.ds(0, L), pl.ds(col0, BPH)], rblk)
        pltpu.sync_copy(cidx_hbm.at[pl.ds(0, L), pl.ds(col0, BPH)], cblk)
        transpose_block(rblk, rT)
        transpose_block(cblk, cT)

        def step(b, fire_next, drain_out):
            s = b & (RING - 1)
            drain(row_cps(b, s))
            fire(col_cps(b, s), add=True)
            if drain_out:
                drain([out_cp(0, s)])
            if fire_next:
                fire(row_cps(b + 1, (b + 1) & (RING - 1)))
            drain(col_cps(b, s))
            fire([out_cp(ph * BPH + b, s)])

        fire(row_cps(0, 0))
        step(0, True, False)
        step(1, True, False)
        lax.fori_loop(2, BPH - 1, lambda b, c: (step(b, True, True), c)[1], 0)
        step(BPH - 1, False, True)
        drain([out_cp(0, (BPH - 2) & (RING - 1))])
        drain([out_cp(0, (BPH - 1) & (RING - 1))])


@jax.jit
def kernel(row_indices, col_indices, row_table, col_table):
    ridxT = row_indices.T.astype(jnp.int32)
    cidxT = col_indices.T.astype(jnp.int32)
    k = pl.kernel(
        _body,
        mesh=plsc.VectorSubcoreMesh(core_axis_name="c", subcore_axis_name="s"),
        compiler_params=pltpu.CompilerParams(
            use_tc_tiling_on_sc=False, needs_layout_passes=False),
        out_type=jax.ShapeDtypeStruct((N, D), jnp.float32),
        scratch_types=[
            pltpu.VMEM((L, BPH), jnp.int32),
            pltpu.VMEM((L, BPH), jnp.int32),
            pltpu.VMEM((BPH, L), jnp.int32),
            pltpu.VMEM((BPH, L), jnp.int32),
            pltpu.VMEM((RING * L, D), jnp.float32),
            pltpu.VMEM_SHARED((1000, D), jnp.float32),
            pltpu.VMEM_SHARED((1000, D), jnp.float32),
            pltpu.SemaphoreType.DMA,
            pltpu.SemaphoreType.DMA,
            pltpu.SemaphoreType.DMA,
        ],
    )
    out = k(ridxT, cidxT, row_table, col_table)
    return out.reshape(B, L, D)
